# Initial kernel scaffold; baseline (speedup 1.0000x reference)
import jax
import jax.numpy as jnp
from jax.experimental import pallas as pl


def kernel(t_in, y_in, alpha_1st, beta_1st, gamma_1st, alpha_2nd, beta_2nd, gamma_2nd, r1_1st, p_1st, r1_2nd, r2_2nd, p_2nd, inds_surf, inds_mant, inds_smt):
    raise NotImplementedError("write your pallas kernel here")


# speedup vs baseline: 10.3648x; 10.3648x over previous
import jax
import jax.numpy as jnp
from jax.experimental import pallas as pl

_B = 512
_S = 1024
_R1 = 8192
_R2 = 24576
_NS = 256
_NM = 256
_NSMT = 4096
_LF = 1e-6
_NAL = 2.0
_EPS = 1e-30

_C1 = 1024
_C2 = 1024
_HI = (_R1 + _R2) // 128

_INTERPRET = False


def _sigmoid(x):
    return 1.0 / (1.0 + jnp.exp(-x))


def _med(t_col):
    Tg = 10.0 + 290.0 * _sigmoid(1e-3 * t_col)
    return jnp.log(Tg / 300.0), 1.0 / Tg


def _p1st_kernel(t_ref, y_ref, a_ref, b_ref, g_ref, r11_ref, p1_ref,
                 surf_ref, mant_ref, smtr_ref, smtc_ref,
                 ra_ref, net_ref, ys_ref, ym_ref, cnt_ref, kmat_ref):
    i = pl.program_id(0)

    @pl.when(i == 0)
    def _init():
        iota_s = jax.lax.broadcasted_iota(jnp.int32, (_S, _NS), 0)
        cnt = jnp.sum((iota_s == surf_ref[...]).astype(jnp.float32),
                      axis=1, keepdims=True)
        cntm = jnp.sum((iota_s == mant_ref[...]).astype(jnp.float32),
                       axis=1, keepdims=True)
        cnt_ref[...] = cnt
        ys_ref[...] = jnp.dot(y_ref[...], cnt,
                              preferred_element_type=jnp.float32)
        ym_ref[...] = jnp.dot(y_ref[...], cntm,
                              preferred_element_type=jnp.float32)
        hi_row = smtr_ref[...] // 128
        lo_col = smtc_ref[...] % 128
        mh = (jax.lax.broadcasted_iota(jnp.int32, (_HI, _NSMT), 0)
              == hi_row).astype(jnp.float32)
        ml = (jax.lax.broadcasted_iota(jnp.int32, (_NSMT, 128), 1)
              == lo_col).astype(jnp.float32)
        kmat_ref[...] = jnp.dot(mh, ml, preferred_element_type=jnp.float32)
        net_ref[...] = jnp.zeros_like(net_ref)

    L, invT = _med(t_ref[...])
    c = a_ref[...] * jnp.exp(b_ref[...] * L - g_ref[...] * invT)
    iota_sub = jax.lax.broadcasted_iota(jnp.int32, (_S, _C1), 0)
    G = (iota_sub == r11_ref[...]).astype(jnp.float32)
    P = (iota_sub == p1_ref[...]).astype(jnp.float32)
    yA = jnp.dot(y_ref[...], G, preferred_element_type=jnp.float32)
    ra = c * yA
    ra_ref[...] = ra
    w = jnp.sum((P - G) * cnt_ref[...], axis=0, keepdims=True)
    net_ref[...] += jnp.sum(ra * w, axis=1, keepdims=True)


def _p2nd_kernel(t_ref, y_ref, a_ref, b_ref, g_ref, r12_ref, r22_ref, p2_ref,
                 cnt_ref, rb_ref, net_ref):
    i = pl.program_id(0)

    @pl.when(i == 0)
    def _init():
        net_ref[...] = jnp.zeros_like(net_ref)

    t = t_ref[...]
    L, invT = _med(t)
    den = jnp.exp(4.0 + 2.0 * jnp.tanh(5e-4 * t))
    c = a_ref[...] * jnp.exp(b_ref[...] * L - g_ref[...] * invT)
    iota_sub = jax.lax.broadcasted_iota(jnp.int32, (_S, _C2), 0)
    Ga = (iota_sub == r12_ref[...]).astype(jnp.float32)
    Gb = (iota_sub == r22_ref[...]).astype(jnp.float32)
    P = (iota_sub == p2_ref[...]).astype(jnp.float32)
    yB1 = jnp.dot(y_ref[...], Ga, preferred_element_type=jnp.float32)
    yB2 = jnp.dot(y_ref[...], Gb, preferred_element_type=jnp.float32)
    rb = c * yB1 * yB2 * den
    rb_ref[...] = rb
    w = jnp.sum((P - Ga - Gb) * cnt_ref[...], axis=0, keepdims=True)
    net_ref[...] += jnp.sum(rb * w, axis=1, keepdims=True)


def _scale(net1, net2, ys, ym):
    nl = _LF * (ys + ym)
    decay = jnp.minimum(_NAL / (nl + _EPS), 1.0)
    sc = decay * _sigmoid(net1 + net2)
    return jnp.log(sc)


def _s1st_kernel(ra_ref, p1_ref, r11_ref, k_ref, n1_ref, n2_ref,
                 ys_ref, ym_ref, dy_ref):
    i = pl.program_id(0)

    @pl.when(i == 0)
    def _init():
        dy_ref[...] = jnp.zeros_like(dy_ref)

    lsc = _scale(n1_ref[...], n2_ref[...], ys_ref[...], ym_ref[...])
    S1 = jnp.exp(k_ref[...] * lsc)
    rs = ra_ref[...] * S1
    iota_lane = jax.lax.broadcasted_iota(jnp.int32, (_C1, _S), 1)
    M = ((iota_lane == p1_ref[...]).astype(jnp.float32)
         - (iota_lane == r11_ref[...]).astype(jnp.float32))
    dy_ref[...] += jnp.dot(rs, M, preferred_element_type=jnp.float32)


def _s2nd_kernel(rb_ref, p2_ref, r12_ref, r22_ref, k_ref, n1_ref, n2_ref,
                 ys_ref, ym_ref, dy1_ref, dy_ref):
    i = pl.program_id(0)

    @pl.when(i == 0)
    def _init():
        dy_ref[...] = dy1_ref[...]

    lsc = _scale(n1_ref[...], n2_ref[...], ys_ref[...], ym_ref[...])
    S2 = jnp.exp(k_ref[...] * lsc)
    rs = rb_ref[...] * S2
    iota_lane = jax.lax.broadcasted_iota(jnp.int32, (_C2, _S), 1)
    M = ((iota_lane == p2_ref[...]).astype(jnp.float32)
         - (iota_lane == r12_ref[...]).astype(jnp.float32)
         - (iota_lane == r22_ref[...]).astype(jnp.float32))
    dy_ref[...] += jnp.dot(rs, M, preferred_element_type=jnp.float32)


def _row(x, n):
    return x.astype(jnp.int32).reshape(1, n)


def _col(x, n):
    return x.astype(jnp.int32).reshape(n, 1)


def kernel(t_in, y_in, alpha_1st, beta_1st, gamma_1st, alpha_2nd, beta_2nd,
           gamma_2nd, r1_1st, p_1st, r1_2nd, r2_2nd, p_2nd,
           inds_surf, inds_mant, inds_smt):
    f32 = jnp.float32
    t_col = t_in.astype(f32).reshape(_B, 1)
    y = y_in.astype(f32)
    a1 = alpha_1st.astype(f32).reshape(1, _R1)
    b1 = beta_1st.astype(f32).reshape(1, _R1)
    g1 = gamma_1st.astype(f32).reshape(1, _R1)
    a2 = alpha_2nd.astype(f32).reshape(1, _R2)
    b2 = beta_2nd.astype(f32).reshape(1, _R2)
    g2 = gamma_2nd.astype(f32).reshape(1, _R2)

    const = lambda *bs: pl.BlockSpec(bs, lambda i: (0,) * len(bs))
    rowblk = lambda c: pl.BlockSpec((1, c), lambda i: (0, i))
    colblk = lambda c: pl.BlockSpec((c, 1), lambda i: (i, 0))

    n1 = _R1 // _C1
    ra, net1, ysurf, ymant, cnt, kmat = pl.pallas_call(
        _p1st_kernel,
        grid=(n1,),
        in_specs=[
            const(_B, 1), const(_B, _S),
            rowblk(_C1), rowblk(_C1), rowblk(_C1),
            rowblk(_C1), rowblk(_C1),
            const(1, _NS), const(1, _NM),
            const(1, _NSMT), const(_NSMT, 1),
        ],
        out_specs=[
            pl.BlockSpec((_B, _C1), lambda i: (0, i)),
            const(_B, 1), const(_B, 1), const(_B, 1),
            const(_S, 1), const(_HI, 128),
        ],
        out_shape=[
            jax.ShapeDtypeStruct((_B, _R1), f32),
            jax.ShapeDtypeStruct((_B, 1), f32),
            jax.ShapeDtypeStruct((_B, 1), f32),
            jax.ShapeDtypeStruct((_B, 1), f32),
            jax.ShapeDtypeStruct((_S, 1), f32),
            jax.ShapeDtypeStruct((_HI, 128), f32),
        ],
        interpret=_INTERPRET,
    )(t_col, y, a1, b1, g1, _row(r1_1st, _R1), _row(p_1st, _R1),
      _row(inds_surf, _NS), _row(inds_mant, _NM),
      _row(inds_smt, _NSMT), _col(inds_smt, _NSMT))

    n2 = _R2 // _C2
    rb, net2 = pl.pallas_call(
        _p2nd_kernel,
        grid=(n2,),
        in_specs=[
            const(_B, 1), const(_B, _S),
            rowblk(_C2), rowblk(_C2), rowblk(_C2),
            rowblk(_C2), rowblk(_C2), rowblk(_C2),
            const(_S, 1),
        ],
        out_specs=[
            pl.BlockSpec((_B, _C2), lambda i: (0, i)),
            const(_B, 1),
        ],
        out_shape=[
            jax.ShapeDtypeStruct((_B, _R2), f32),
            jax.ShapeDtypeStruct((_B, 1), f32),
        ],
        interpret=_INTERPRET,
    )(t_col, y, a2, b2, g2, _row(r1_2nd, _R2), _row(r2_2nd, _R2),
      _row(p_2nd, _R2), cnt)

    k_row = kmat.reshape(1, _R1 + _R2)
    k1 = k_row[:, :_R1]
    k2 = k_row[:, _R1:]

    dy1 = pl.pallas_call(
        _s1st_kernel,
        grid=(n1,),
        in_specs=[
            pl.BlockSpec((_B, _C1), lambda i: (0, i)),
            colblk(_C1), colblk(_C1), rowblk(_C1),
            const(_B, 1), const(_B, 1), const(_B, 1), const(_B, 1),
        ],
        out_specs=const(_B, _S),
        out_shape=jax.ShapeDtypeStruct((_B, _S), f32),
        interpret=_INTERPRET,
    )(ra, _col(p_1st, _R1), _col(r1_1st, _R1), k1, net1, net2, ysurf, ymant)

    dy = pl.pallas_call(
        _s2nd_kernel,
        grid=(n2,),
        in_specs=[
            pl.BlockSpec((_B, _C2), lambda i: (0, i)),
            colblk(_C2), colblk(_C2), colblk(_C2), rowblk(_C2),
            const(_B, 1), const(_B, 1), const(_B, 1), const(_B, 1),
            const(_B, _S),
        ],
        out_specs=const(_B, _S),
        out_shape=jax.ShapeDtypeStruct((_B, _S), f32),
        interpret=_INTERPRET,
    )(rb, _col(p_2nd, _R2), _col(r1_2nd, _R2), _col(r2_2nd, _R2), k2,
      net1, net2, ysurf, ymant, dy1)

    return dy
